# Initial kernel scaffold; baseline (speedup 1.0000x reference)
import jax
import jax.numpy as jnp
from jax.experimental import pallas as pl


def kernel(encoded_parents, masks, kmer_embedding):
    raise NotImplementedError("write your pallas kernel here")


# speedup vs baseline: 171.0980x; 171.0980x over previous
import functools

import jax
import jax.numpy as jnp
from jax import lax
from jax.experimental import pallas as pl
from jax.experimental.pallas import tpu as pltpu
from jax.experimental.pallas import tpu_sc as plsc

_B, _L = 16384, 200
_N = _B * _L
_NC, _NS = 2, 16
_NW = _NC * _NS
_PER_W = _N // _NW
_CHUNK = 12800
_NCHUNK = _PER_W // _CHUNK
_TBL = 1024
_LANES = 16


def _body(table_hbm, idx_hbm, out_hbm, etab_v, idx_v, out_v):
    wid = lax.axis_index("s") * _NC + lax.axis_index("c")
    base = wid * _PER_W

    pltpu.sync_copy(table_hbm, etab_v)

    def expb(j, carry):
        sl = pl.ds(j * _LANES, _LANES)
        etab_v[sl] = jnp.exp(etab_v[sl])
        return carry

    lax.fori_loop(0, _TBL // _LANES, expb, 0)

    for c in range(_NCHUNK):
        off = base + c * _CHUNK
        pltpu.sync_copy(idx_hbm.at[pl.ds(off, _CHUNK)], idx_v)

        def gb(i, carry):
            sl = pl.ds(i * _LANES, _LANES)
            out_v[sl] = plsc.load_gather(etab_v, [idx_v[sl]])
            return carry

        lax.fori_loop(0, _CHUNK // _LANES, gb, 0)
        pltpu.sync_copy(out_v, out_hbm.at[pl.ds(off, _CHUNK)])


@jax.jit
def _run(table_flat, idx_flat):
    mesh = plsc.VectorSubcoreMesh(core_axis_name="c", subcore_axis_name="s")
    f = pl.kernel(
        _body,
        out_type=jax.ShapeDtypeStruct((_N,), jnp.float32),
        mesh=mesh,
        scratch_types=[
            pltpu.VMEM((_TBL,), jnp.float32),
            pltpu.VMEM((_CHUNK,), jnp.int32),
            pltpu.VMEM((_CHUNK,), jnp.float32),
        ],
        compiler_params=pltpu.CompilerParams(needs_layout_passes=False),
    )
    return f(table_flat, idx_flat)


def kernel(encoded_parents, masks, kmer_embedding):
    del masks
    idx = encoded_parents.reshape(-1)
    tab = kmer_embedding.reshape(-1)
    out = _run(tab, idx)
    return out.reshape(encoded_parents.shape)


# speedup vs baseline: 236.6112x; 1.3829x over previous
import functools

import jax
import jax.numpy as jnp
from jax import lax
from jax.experimental import pallas as pl
from jax.experimental.pallas import tpu as pltpu
from jax.experimental.pallas import tpu_sc as plsc

_B, _L = 16384, 200
_N = _B * _L
_NC, _NS = 2, 16
_NW = _NC * _NS
_PER_W = _N // _NW
_CHUNK = 25600
_NCHUNK = _PER_W // _CHUNK
_TBL = 1024
_LANES = 16


def _body(table_hbm, idx_hbm, out_hbm,
          etab_v, idx0, idx1, out0, out1, si0, si1, so0, so1):
    wid = lax.axis_index("s") * _NC + lax.axis_index("c")
    base = wid * _PER_W
    idx_b, out_b, si, so = (idx0, idx1), (out0, out1), (si0, si1), (so0, so1)

    pend_in = {}
    pend_out = {}
    pend_in[0] = pltpu.async_copy(
        idx_hbm.at[pl.ds(base, _CHUNK)], idx0, si0)

    pltpu.sync_copy(table_hbm, etab_v)

    def expb(j, carry):
        sl = pl.ds(j * _LANES, _LANES)
        etab_v[sl] = jnp.exp(etab_v[sl])
        return carry

    lax.fori_loop(0, _TBL // _LANES, expb, 0)

    for c in range(_NCHUNK):
        b = c & 1
        if c + 1 < _NCHUNK:
            pend_in[c + 1] = pltpu.async_copy(
                idx_hbm.at[pl.ds(base + (c + 1) * _CHUNK, _CHUNK)],
                idx_b[1 - b], si[1 - b])
        pend_in[c].wait()
        if c >= 2:
            pend_out[c - 2].wait()

        @plsc.parallel_loop(0, _CHUNK, _LANES, unroll=8)
        def gb(i, _ib=idx_b[b], _ob=out_b[b]):
            sl = pl.ds(i, _LANES)
            _ob[sl] = plsc.load_gather(etab_v, [_ib[sl]])

        pend_out[c] = pltpu.async_copy(
            out_b[b], out_hbm.at[pl.ds(base + c * _CHUNK, _CHUNK)], so[b])

    pend_out[_NCHUNK - 2].wait()
    pend_out[_NCHUNK - 1].wait()


@jax.jit
def _run(table_flat, idx_flat):
    mesh = plsc.VectorSubcoreMesh(core_axis_name="c", subcore_axis_name="s")
    f = pl.kernel(
        _body,
        out_type=jax.ShapeDtypeStruct((_N,), jnp.float32),
        mesh=mesh,
        scratch_types=[
            pltpu.VMEM((_TBL,), jnp.float32),
            pltpu.VMEM((_CHUNK,), jnp.int32),
            pltpu.VMEM((_CHUNK,), jnp.int32),
            pltpu.VMEM((_CHUNK,), jnp.float32),
            pltpu.VMEM((_CHUNK,), jnp.float32),
            pltpu.SemaphoreType.DMA,
            pltpu.SemaphoreType.DMA,
            pltpu.SemaphoreType.DMA,
            pltpu.SemaphoreType.DMA,
        ],
        compiler_params=pltpu.CompilerParams(needs_layout_passes=False),
    )
    return f(table_flat, idx_flat)


def kernel(encoded_parents, masks, kmer_embedding):
    del masks
    idx = encoded_parents.reshape(-1)
    tab = kmer_embedding.reshape(-1)
    out = _run(tab, idx)
    return out.reshape(encoded_parents.shape)


# speedup vs baseline: 377.3408x; 1.5948x over previous
import jax
import jax.numpy as jnp
from jax import lax
from jax.experimental import pallas as pl
from jax.experimental.pallas import tpu as pltpu
from jax.experimental.pallas import tpu_sc as plsc

_B, _L = 16384, 200
_NC, _NS = 2, 16
_NW = _NC * _NS
_ROWS_W = _B // _NW
_RCHUNK = 64
_NCHUNK = _ROWS_W // _RCHUNK
_TBL = 1024
_LANES = 16
_STEPS = _RCHUNK * _L // _LANES
_CPR = _L // 8


def _body(table_hbm, idx_hbm, out_hbm,
          etab_v, idx0, idx1, out0, out1, si0, si1, so0, so1):
    wid = lax.axis_index("s") * _NC + lax.axis_index("c")
    row_base = wid * _ROWS_W
    idx_b, out_b, si, so = (idx0, idx1), (out0, out1), (si0, si1), (so0, so1)

    pend_in = {}
    pend_out = {}
    pend_in[0] = pltpu.async_copy(
        idx_hbm.at[pl.ds(row_base, _RCHUNK), :], idx0, si0)

    pltpu.sync_copy(table_hbm, etab_v)
    lane = lax.iota(jnp.int32, _LANES)

    def expb(j, carry):
        sl = pl.ds(j * _LANES, _LANES)
        etab_v[sl] = jnp.exp(etab_v[sl])
        return carry

    lax.fori_loop(0, _TBL // _LANES, expb, 0)

    lane_r = lane // 8
    lane_c = lane % 8

    for c in range(_NCHUNK):
        b = c & 1
        if c + 1 < _NCHUNK:
            pend_in[c + 1] = pltpu.async_copy(
                idx_hbm.at[pl.ds(row_base + (c + 1) * _RCHUNK, _RCHUNK), :],
                idx_b[1 - b], si[1 - b])
        pend_in[c].wait()
        if c >= 2:
            pend_out[c - 2].wait()

        @plsc.parallel_loop(0, _STEPS, 1, unroll=8)
        def gb(p, _ib=idx_b[b], _ob=out_b[b]):
            rvec = (p // _CPR) * 2 + lane_r
            cvec = (p % _CPR) * 8 + lane_c
            iv = plsc.load_gather(_ib, [rvec, cvec])
            vals = plsc.load_gather(etab_v, [iv])
            plsc.store_scatter(_ob, [rvec, cvec], vals)

        pend_out[c] = pltpu.async_copy(
            out_b[b], out_hbm.at[pl.ds(row_base + c * _RCHUNK, _RCHUNK), :],
            so[b])

    pend_out[_NCHUNK - 2].wait()
    pend_out[_NCHUNK - 1].wait()


@jax.jit
def _run(table, idx):
    mesh = plsc.VectorSubcoreMesh(core_axis_name="c", subcore_axis_name="s")
    f = pl.kernel(
        _body,
        out_type=jax.ShapeDtypeStruct((_B, _L), jnp.float32),
        mesh=mesh,
        scratch_types=[
            pltpu.VMEM((_TBL,), jnp.float32),
            pltpu.VMEM((_RCHUNK, _L), jnp.int32),
            pltpu.VMEM((_RCHUNK, _L), jnp.int32),
            pltpu.VMEM((_RCHUNK, _L), jnp.float32),
            pltpu.VMEM((_RCHUNK, _L), jnp.float32),
            pltpu.SemaphoreType.DMA,
            pltpu.SemaphoreType.DMA,
            pltpu.SemaphoreType.DMA,
            pltpu.SemaphoreType.DMA,
        ],
        compiler_params=pltpu.CompilerParams(needs_layout_passes=False),
    )
    return f(table, idx)


def kernel(encoded_parents, masks, kmer_embedding):
    del masks
    return _run(kmer_embedding.reshape(-1), encoded_parents)


# speedup vs baseline: 768.4065x; 2.0364x over previous
import jax
import jax.numpy as jnp
from jax import lax
from jax.experimental import pallas as pl
from jax.experimental.pallas import tpu as pltpu
from jax.experimental.pallas import tpu_sc as plsc

_B, _L = 16384, 200
_NC, _NS = 2, 16
_NW = _NC * _NS
_COLS_W = _B // _NW
_CCHUNK = 128
_NCHUNK = _COLS_W // _CCHUNK
_TBL = 1024
_LANES = 16
_SEG = _CCHUNK // _LANES


def _body(table_hbm, idx_hbm, out_hbm,
          etab_v, idx0, idx1, out0, out1, si0, si1, so0, so1):
    wid = lax.axis_index("s") * _NC + lax.axis_index("c")
    col_base = wid * _COLS_W
    idx_b, out_b, si, so = (idx0, idx1), (out0, out1), (si0, si1), (so0, so1)

    pend_in = {}
    pend_out = {}
    pend_in[0] = pltpu.async_copy(
        idx_hbm.at[:, pl.ds(col_base, _CCHUNK)], idx0, si0)

    pltpu.sync_copy(table_hbm, etab_v)

    def expb(j, carry):
        sl = pl.ds(j * _LANES, _LANES)
        etab_v[sl] = jnp.exp(etab_v[sl])
        return carry

    lax.fori_loop(0, _TBL // _LANES, expb, 0)

    for c in range(_NCHUNK):
        b = c & 1
        if c + 1 < _NCHUNK:
            pend_in[c + 1] = pltpu.async_copy(
                idx_hbm.at[:, pl.ds(col_base + (c + 1) * _CCHUNK, _CCHUNK)],
                idx_b[1 - b], si[1 - b])
        pend_in[c].wait()
        if c >= 2:
            pend_out[c - 2].wait()

        @plsc.parallel_loop(0, _L, 1, unroll=2)
        def gb(r, _ib=idx_b[b], _ob=out_b[b]):
            for u in range(_SEG):
                sl = pl.ds(u * _LANES, _LANES)
                _ob[r, sl] = plsc.load_gather(etab_v, [_ib[r, sl]])

        pend_out[c] = pltpu.async_copy(
            out_b[b],
            out_hbm.at[:, pl.ds(col_base + c * _CCHUNK, _CCHUNK)], so[b])

    pend_out[_NCHUNK - 2].wait()
    pend_out[_NCHUNK - 1].wait()


@jax.jit
def _run(table, idx_t):
    mesh = plsc.VectorSubcoreMesh(core_axis_name="c", subcore_axis_name="s")
    f = pl.kernel(
        _body,
        out_type=jax.ShapeDtypeStruct((_L, _B), jnp.float32),
        mesh=mesh,
        scratch_types=[
            pltpu.VMEM((_TBL,), jnp.float32),
            pltpu.VMEM((_L, _CCHUNK), jnp.int32),
            pltpu.VMEM((_L, _CCHUNK), jnp.int32),
            pltpu.VMEM((_L, _CCHUNK), jnp.float32),
            pltpu.VMEM((_L, _CCHUNK), jnp.float32),
            pltpu.SemaphoreType.DMA,
            pltpu.SemaphoreType.DMA,
            pltpu.SemaphoreType.DMA,
            pltpu.SemaphoreType.DMA,
        ],
        compiler_params=pltpu.CompilerParams(needs_layout_passes=False),
    )
    return f(table, idx_t).T


def kernel(encoded_parents, masks, kmer_embedding):
    del masks
    return _run(kmer_embedding.reshape(-1), encoded_parents.T)
